# baseline (speedup 1.0000x reference)
import functools

import jax
import jax.numpy as jnp
from jax import lax
from jax.experimental import pallas as pl
from jax.experimental.pallas import tpu as pltpu
from jax.experimental.pallas import tpu_sc as plsc


@functools.lru_cache(maxsize=None)
def _make_kernel(B, D, VU, VJ, VS, VC):
    info = plsc.get_sparse_core_info()
    NC, NS, L = info.num_cores, info.num_subcores, info.num_lanes
    NW = NC * NS
    G = 128 // D
    assert D * G == 128 and VU % G == 0 and VJ % G == 0
    assert B % NW == 0
    bpw = B // NW
    cs = min(bpw, 128)
    assert bpw % cs == 0 and cs % L == 0
    mesh = plsc.VectorSubcoreMesh(core_axis_name="c", subcore_axis_name="s")

    @functools.partial(
        pl.kernel,
        mesh=mesh,
        out_type=jax.ShapeDtypeStruct((B, 4 * D), jnp.float32),
        scratch_types=[
            pltpu.VMEM((cs,), jnp.int32),
            pltpu.VMEM((cs,), jnp.int32),
            pltpu.VMEM((cs,), jnp.int32),
            pltpu.VMEM((cs,), jnp.int32),
            pltpu.VMEM((cs,), jnp.int32),
            pltpu.VMEM((cs,), jnp.int32),
            pltpu.VMEM((cs, G * D), jnp.float32),
            pltpu.VMEM((cs, G * D), jnp.float32),
            pltpu.VMEM((VS, D), jnp.float32),
            pltpu.VMEM((VC, D), jnp.float32),
            pltpu.VMEM((cs, 4 * D), jnp.float32),
            pltpu.SemaphoreType.DMA,
        ],
        compiler_params=pltpu.CompilerParams(needs_layout_passes=False),
    )
    def k(email_hbm, job_hbm, size_hbm, country_hbm,
          user_t, job_t, size_t, country_t, out_hbm,
          iu, ij, isz, ic, gu, gj, ru, rj, ts, tc, stage, sem):
        wid = lax.axis_index("s") * NC + lax.axis_index("c")
        pltpu.sync_copy(size_t, ts)
        pltpu.sync_copy(country_t, tc)
        def chunk_body(ch, carry):
            base = wid * bpw + ch * cs
            pltpu.sync_copy(email_hbm.at[pl.ds(base, cs)], iu)
            pltpu.sync_copy(job_hbm.at[pl.ds(base, cs)], ij)
            pltpu.sync_copy(size_hbm.at[pl.ds(base, cs)], isz)
            pltpu.sync_copy(country_hbm.at[pl.ds(base, cs)], ic)
            for g in range(cs // L):
                sl = pl.ds(g * L, L)
                gu[sl] = iu[sl] >> 2
                gj[sl] = ij[sl] >> 2
            cu = pltpu.async_copy(user_t.at[gu], ru, sem)
            cj = pltpu.async_copy(job_t.at[gj], rj, sem)
            iota = lax.iota(jnp.int32, L)
            cu.wait()
            cj.wait()
            for g in range(cs // L):
                sl = pl.ds(g * L, L)
                jvec = g * L + iota
                ucol = (iu[sl] & (G - 1)) * D
                jcol = (ij[sl] & (G - 1)) * D
                srow = isz[sl]
                crow = ic[sl]
                for c in range(D):
                    cvec = jnp.full((L,), c, jnp.int32)
                    plsc.store_scatter(
                        stage, [jvec, cvec],
                        plsc.load_gather(ru, [jvec, ucol + c]))
                    plsc.store_scatter(
                        stage, [jvec, cvec + D],
                        plsc.load_gather(rj, [jvec, jcol + c]))
                    plsc.store_scatter(
                        stage, [jvec, cvec + 2 * D],
                        plsc.load_gather(ts, [srow, cvec]))
                    plsc.store_scatter(
                        stage, [jvec, cvec + 3 * D],
                        plsc.load_gather(tc, [crow, cvec]))
            pltpu.sync_copy(stage, out_hbm.at[pl.ds(base, cs)])
            return carry

        lax.fori_loop(0, bpw // cs, chunk_body, 0)

    return k


def kernel(email_address, job_title, company_size, country,
           user_table, job_table, size_table, country_table):
    B = email_address.shape[0]
    VU, D = user_table.shape
    VJ = job_table.shape[0]
    VS = size_table.shape[0]
    VC = country_table.shape[0]
    k = _make_kernel(B, D, VU, VJ, VS, VC)
    return k(email_address, job_title, company_size, country,
             user_table.reshape(VU // 4, 4 * D),
             job_table.reshape(VJ // 4, 4 * D),
             size_table, country_table)


# speedup vs baseline: 1.0414x; 1.0414x over previous
import functools

import jax
import jax.numpy as jnp
from jax import lax
from jax.experimental import pallas as pl
from jax.experimental.pallas import tpu as pltpu
from jax.experimental.pallas import tpu_sc as plsc

_CHUNK = 128


@functools.lru_cache(maxsize=None)
def _make_kernel(B, D, VU, VJ, VS, VC):
    info = plsc.get_sparse_core_info()
    NC, NS = info.num_cores, info.num_subcores
    NW = NC * NS
    assert B % (NW * _CHUNK) == 0
    bpw = B // NW
    nch = bpw // _CHUNK
    mesh = plsc.VectorSubcoreMesh(core_axis_name="c", subcore_axis_name="s")

    idx_t = pltpu.VMEM((nch, _CHUNK), jnp.int32)
    row_t = pltpu.VMEM((bpw, D), jnp.float32)

    @functools.partial(
        pl.kernel,
        mesh=mesh,
        out_type=jax.ShapeDtypeStruct((B, 4 * D), jnp.float32),
        scratch_types=[idx_t, idx_t, idx_t, idx_t,
                       row_t, row_t, row_t, row_t,
                       pltpu.SemaphoreType.DMA],
        compiler_params=pltpu.CompilerParams(use_tc_tiling_on_sc=False),
    )
    def k(email_hbm, job_hbm, size_hbm, country_hbm,
          user_t, job_t, size_t, country_t, out_hbm,
          iu, ij, isz, ic, ru, rj, rs, rc, sem):
        wid = lax.axis_index("s") * NC + lax.axis_index("c")
        row0 = wid * nch
        pltpu.sync_copy(email_hbm.at[pl.ds(row0, nch)], iu)
        pltpu.sync_copy(job_hbm.at[pl.ds(row0, nch)], ij)
        pltpu.sync_copy(size_hbm.at[pl.ds(row0, nch)], isz)
        pltpu.sync_copy(country_hbm.at[pl.ds(row0, nch)], ic)
        copies = []
        for j in range(nch):
            sl = pl.ds(j * _CHUNK, _CHUNK)
            copies.append(pltpu.async_copy(user_t.at[iu.at[j]], ru.at[sl], sem))
            copies.append(pltpu.async_copy(job_t.at[ij.at[j]], rj.at[sl], sem))
            copies.append(pltpu.async_copy(size_t.at[isz.at[j]], rs.at[sl], sem))
            copies.append(pltpu.async_copy(country_t.at[ic.at[j]], rc.at[sl], sem))
        for c in copies:
            c.wait()
        base = wid * bpw
        rows = pl.ds(base, bpw)
        pltpu.sync_copy(ru, out_hbm.at[rows, pl.ds(0 * D, D)])
        pltpu.sync_copy(rj, out_hbm.at[rows, pl.ds(1 * D, D)])
        pltpu.sync_copy(rs, out_hbm.at[rows, pl.ds(2 * D, D)])
        pltpu.sync_copy(rc, out_hbm.at[rows, pl.ds(3 * D, D)])

    return k


def kernel(email_address, job_title, company_size, country,
           user_table, job_table, size_table, country_table):
    B = email_address.shape[0]
    VU, D = user_table.shape
    VJ = job_table.shape[0]
    VS = size_table.shape[0]
    VC = country_table.shape[0]
    k = _make_kernel(B, D, VU, VJ, VS, VC)
    m = B // _CHUNK
    return k(email_address.reshape(m, _CHUNK), job_title.reshape(m, _CHUNK),
             company_size.reshape(m, _CHUNK), country.reshape(m, _CHUNK),
             user_table, job_table, size_table, country_table)
